# Initial kernel scaffold; baseline (speedup 1.0000x reference)
import jax
import jax.numpy as jnp
from jax.experimental import pallas as pl


def kernel(x, edge_index, degree_embedding):
    raise NotImplementedError("write your pallas kernel here")


# speedup vs baseline: 1.0334x; 1.0334x over previous
import functools

import jax
import jax.numpy as jnp
from jax import lax
from jax.experimental import pallas as pl
from jax.experimental.pallas import tpu as pltpu
from jax.experimental.pallas import tpu_sc as plsc

NC = 2
NS = 16
NW = NC * NS
LANES = 16

CH = 160
HALF = CH // 2


def _worker_id():
    return lax.axis_index("s") * NC + lax.axis_index("c")


def _hist_body(n_edges, num_nodes, idx_chunk, dst_hbm, hists_hbm, hist_v, idx_v):
    wid = _worker_id()
    epw = n_edges // NW
    base = wid * epw

    def zero(i, _):
        hist_v[pl.ds(i * LANES, LANES)] = jnp.zeros((LANES,), jnp.int32)
        return _

    lax.fori_loop(0, num_nodes // LANES, zero, None)

    ones = jnp.ones((LANES,), jnp.int32)
    for c in range(epw // idx_chunk):
        pltpu.sync_copy(dst_hbm.at[pl.ds(base + c * idx_chunk, idx_chunk)], idx_v)

        def acc(j, _):
            idx = idx_v[pl.ds(j * LANES, LANES)]
            plsc.addupdate_scatter(hist_v, [idx], ones)
            return _

        lax.fori_loop(0, idx_chunk // LANES, acc, None)

    pltpu.sync_copy(hist_v, hists_hbm.at[wid])


def _gather_body(num_nodes, x_hbm, hists_hbm, emb_hbm, out_hbm,
                 part_v, deg_v, rows_v, x_v, sem, sem_x):
    wid = _worker_id()
    nchunks = num_nodes // CH
    nfull = nchunks // NW
    extra = nchunks - nfull * NW
    nc = nfull + jnp.where(wid < extra, 1, 0)

    def chunk(t, _):
        c = wid + t * NW
        node0 = c * CH
        x_cp = pltpu.async_copy(x_hbm.at[pl.ds(node0, CH)], x_v, sem_x)
        pltpu.sync_copy(hists_hbm.at[:, pl.ds(node0, CH)], part_v)
        for j in range(CH // LANES):
            acc0 = part_v[0, pl.ds(j * LANES, LANES)]
            for p in range(1, NW):
                acc0 = acc0 + part_v[p, pl.ds(j * LANES, LANES)]
            acc0 = jnp.minimum(acc0, num_nodes - 1)
            deg_v[(j * LANES) // HALF, pl.ds((j * LANES) % HALF, LANES)] = acc0
        cp0 = pltpu.async_copy(emb_hbm.at[deg_v.at[0]], rows_v.at[pl.ds(0, HALF)], sem)
        cp1 = pltpu.async_copy(emb_hbm.at[deg_v.at[1]], rows_v.at[pl.ds(HALF, HALF)], sem)
        cp0.wait()
        cp1.wait()
        x_cp.wait()

        def add_row(r, _):
            for k in range(NODE_DIM_VECS):
                x_v[r, pl.ds(k * LANES, LANES)] = (
                    x_v[r, pl.ds(k * LANES, LANES)]
                    + rows_v[r, pl.ds(k * LANES, LANES)]
                )
            return _

        lax.fori_loop(0, CH, add_row, None)
        pltpu.sync_copy(x_v, out_hbm.at[pl.ds(node0, CH)])
        return _

    lax.fori_loop(0, nc, chunk, None)


NODE_DIM_VECS = 128 // LANES


@jax.jit
def kernel(x, edge_index, degree_embedding):
    num_nodes, node_dim = x.shape
    n_edges = edge_index.shape[1]
    assert node_dim == 128 and num_nodes % CH == 0 and n_edges % (NW * LANES) == 0

    dst = edge_index[1]
    mesh = plsc.VectorSubcoreMesh(core_axis_name="c", subcore_axis_name="s")

    idx_chunk = 10000
    hist_call = pl.kernel(
        functools.partial(_hist_body, n_edges, num_nodes, idx_chunk),
        out_type=jax.ShapeDtypeStruct((NW, num_nodes), jnp.int32),
        mesh=mesh,
        scratch_types=[
            pltpu.VMEM((num_nodes,), jnp.int32),
            pltpu.VMEM((idx_chunk,), jnp.int32),
        ],
        compiler_params=pltpu.CompilerParams(needs_layout_passes=False, use_tc_tiling_on_sc=False),
    )
    hists = hist_call(dst)

    gather_call = pl.kernel(
        functools.partial(_gather_body, num_nodes),
        out_type=jax.ShapeDtypeStruct((num_nodes, node_dim), jnp.float32),
        mesh=mesh,
        scratch_types=[
            pltpu.VMEM((NW, CH), jnp.int32),
            pltpu.VMEM((2, HALF), jnp.int32),
            pltpu.VMEM((CH, node_dim), jnp.float32),
            pltpu.VMEM((CH, node_dim), jnp.float32),
            pltpu.SemaphoreType.DMA,
            pltpu.SemaphoreType.DMA,
        ],
        compiler_params=pltpu.CompilerParams(needs_layout_passes=False, use_tc_tiling_on_sc=False),
    )
    return gather_call(x, hists, degree_embedding)


# speedup vs baseline: 1.0543x; 1.0202x over previous
import functools

import jax
import jax.numpy as jnp
from jax import lax
from jax.experimental import pallas as pl
from jax.experimental.pallas import tpu as pltpu
from jax.experimental.pallas import tpu_sc as plsc

NC = 2
NS = 16
NW = NC * NS
LANES = 16

CH = 160
HALF = CH // 2
IDX_CHUNK = 10000


def _worker_id():
    return lax.axis_index("s") * NC + lax.axis_index("c")


def _hist_body(n_edges, num_nodes, dst_hbm, hists_hbm,
               hist_v, idx0, idx1, sem0, sem1):
    wid = _worker_id()
    epw = n_edges // NW
    base = wid * epw
    nchk = epw // IDX_CHUNK
    idx_v = (idx0, idx1)
    sems = (sem0, sem1)

    zvec = jnp.zeros((LANES,), jnp.int32)

    def zero(i, _):
        for u in range(10):
            hist_v[pl.ds((i * 10 + u) * LANES, LANES)] = zvec
        return _

    lax.fori_loop(0, num_nodes // (10 * LANES), zero, None)

    ones = jnp.ones((LANES,), jnp.int32)
    cps = {}
    cps[0] = pltpu.async_copy(
        dst_hbm.at[pl.ds(base, IDX_CHUNK)], idx0, sem0)
    for c in range(nchk):
        b = c & 1
        if c + 1 < nchk:
            cps[1 - b] = pltpu.async_copy(
                dst_hbm.at[pl.ds(base + (c + 1) * IDX_CHUNK, IDX_CHUNK)],
                idx_v[1 - b], sems[1 - b])
        cps[b].wait()
        buf = idx_v[b]

        def acc(j, _):
            for u in range(5):
                idx = buf[pl.ds((j * 5 + u) * LANES, LANES)]
                plsc.addupdate_scatter(hist_v, [idx], ones)
            return _

        lax.fori_loop(0, IDX_CHUNK // (5 * LANES), acc, None)

    pltpu.sync_copy(hist_v, hists_hbm.at[wid])


def _gather_body(num_nodes, x_hbm, hists_hbm, emb_hbm, out_hbm,
                 part0, part1, deg0, deg1, rows0, rows1, xb0, xb1,
                 sp0, sp1, sx0, sx1, sr0, sr1, so0, so1):
    wid = _worker_id()
    nchunks = num_nodes // CH
    nct = nchunks // NW + (1 if nchunks % NW else 0)
    part_v = (part0, part1)
    deg_v = (deg0, deg1)
    rows_v = (rows0, rows1)
    x_v = (xb0, xb1)
    sem_p = (sp0, sp1)
    sem_x = (sx0, sx1)
    sem_r = (sr0, sr1)
    sem_o = (so0, so1)

    def node0_of(t):
        return jnp.minimum(wid + t * NW, nchunks - 1) * CH

    def issue_in(t, b):
        n0 = node0_of(t)
        cp_p = pltpu.async_copy(hists_hbm.at[:, pl.ds(n0, CH)], part_v[b], sem_p[b])
        cp_x = pltpu.async_copy(x_hbm.at[pl.ds(n0, CH)], x_v[b], sem_x[b])
        return cp_p, cp_x

    cps = {("in", 0): issue_in(0, 0)}
    for t in range(nct):
        b = t & 1
        if t + 1 < nct:
            if t >= 1:
                cps[("out", 1 - b)].wait()
            cps[("in", 1 - b)] = issue_in(t + 1, 1 - b)
        cp_p, cp_x = cps[("in", b)]
        cp_p.wait()
        part = part_v[b]
        deg = deg_v[b]

        def merge(j, _):
            acc0 = part[0, pl.ds(j * LANES, LANES)]
            for p in range(1, NW):
                acc0 = acc0 + part[p, pl.ds(j * LANES, LANES)]
            deg[pl.ds(j * LANES, LANES)] = jnp.minimum(acc0, num_nodes - 1)
            return _

        lax.fori_loop(0, CH // LANES, merge, None)
        g0 = pltpu.async_copy(
            emb_hbm.at[deg.at[pl.ds(0, HALF)]], rows_v[b].at[pl.ds(0, HALF)], sem_r[b])
        g1 = pltpu.async_copy(
            emb_hbm.at[deg.at[pl.ds(HALF, HALF)]], rows_v[b].at[pl.ds(HALF, HALF)], sem_r[b])
        cp_x.wait()
        g0.wait()
        g1.wait()
        xb = x_v[b]
        rows = rows_v[b]

        def add_row(r, _):
            for k in range(128 // LANES):
                xb[r, pl.ds(k * LANES, LANES)] = (
                    xb[r, pl.ds(k * LANES, LANES)]
                    + rows[r, pl.ds(k * LANES, LANES)]
                )
            return _

        lax.fori_loop(0, CH, add_row, None)
        cps[("out", b)] = pltpu.async_copy(
            xb, out_hbm.at[pl.ds(node0_of(t), CH)], sem_o[b])
    cps[("out", 0)].wait()
    cps[("out", 1)].wait()


@jax.jit
def kernel(x, edge_index, degree_embedding):
    num_nodes, node_dim = x.shape
    n_edges = edge_index.shape[1]
    assert node_dim == 128 and num_nodes % CH == 0
    assert n_edges % (NW * IDX_CHUNK) == 0

    dst = edge_index[1]
    mesh = plsc.VectorSubcoreMesh(core_axis_name="c", subcore_axis_name="s")
    params = pltpu.CompilerParams(
        needs_layout_passes=False, use_tc_tiling_on_sc=False)

    hist_call = pl.kernel(
        functools.partial(_hist_body, n_edges, num_nodes),
        out_type=jax.ShapeDtypeStruct((NW, num_nodes), jnp.int32),
        mesh=mesh,
        scratch_types=[
            pltpu.VMEM((num_nodes,), jnp.int32),
            pltpu.VMEM((IDX_CHUNK,), jnp.int32),
            pltpu.VMEM((IDX_CHUNK,), jnp.int32),
            pltpu.SemaphoreType.DMA,
            pltpu.SemaphoreType.DMA,
        ],
        compiler_params=params,
    )
    hists = hist_call(dst)

    gather_call = pl.kernel(
        functools.partial(_gather_body, num_nodes),
        out_type=jax.ShapeDtypeStruct((num_nodes, node_dim), jnp.float32),
        mesh=mesh,
        scratch_types=(
            [pltpu.VMEM((NW, CH), jnp.int32)] * 2
            + [pltpu.VMEM((CH,), jnp.int32)] * 2
            + [pltpu.VMEM((CH, node_dim), jnp.float32)] * 4
            + [pltpu.SemaphoreType.DMA] * 8
        ),
        compiler_params=params,
    )
    return gather_call(x, hists, degree_embedding)


# speedup vs baseline: 1.0698x; 1.0147x over previous
import functools

import jax
import jax.numpy as jnp
from jax import lax
from jax.experimental import pallas as pl
from jax.experimental.pallas import tpu as pltpu
from jax.experimental.pallas import tpu_sc as plsc

NC = 2
NS = 16
NW = NC * NS
LANES = 16

CH = 160
HALF = CH // 2
IDX_CHUNK = 10000


def _worker_id():
    return lax.axis_index("s") * NC + lax.axis_index("c")


def _hist_body(n_edges, num_nodes, dst_hbm, hists_hbm,
               hist_v, idx0, idx1, sem0, sem1):
    wid = _worker_id()
    epw = n_edges // NW
    base = wid * epw
    nchk = epw // IDX_CHUNK
    idx_v = (idx0, idx1)
    sems = (sem0, sem1)

    zvec = jnp.zeros((LANES,), jnp.int32)

    @plsc.parallel_loop(0, num_nodes // LANES, unroll=8)
    def _zero(i):
        hist_v[pl.ds(i * LANES, LANES)] = zvec

    ones = jnp.ones((LANES,), jnp.int32)
    cps = {}
    cps[0] = pltpu.async_copy(
        dst_hbm.at[pl.ds(base, IDX_CHUNK)], idx0, sem0)
    for c in range(nchk):
        b = c & 1
        if c + 1 < nchk:
            cps[1 - b] = pltpu.async_copy(
                dst_hbm.at[pl.ds(base + (c + 1) * IDX_CHUNK, IDX_CHUNK)],
                idx_v[1 - b], sems[1 - b])
        cps[b].wait()
        buf = idx_v[b]

        @plsc.parallel_loop(0, IDX_CHUNK // LANES, unroll=8)
        def _acc(j):
            idx = buf[pl.ds(j * LANES, LANES)]
            plsc.addupdate_scatter(hist_v, [idx], ones)

    pltpu.sync_copy(hist_v, hists_hbm.at[wid])


def _gather_body(num_nodes, x_hbm, hists_hbm, emb_hbm, out_hbm,
                 part0, part1, deg0, deg1, rows0, rows1, xb0, xb1,
                 sp0, sp1, sx0, sx1, sr0, sr1, so0, so1):
    wid = _worker_id()
    nchunks = num_nodes // CH
    nct = nchunks // NW + (1 if nchunks % NW else 0)
    part_v = (part0, part1)
    deg_v = (deg0, deg1)
    rows_v = (rows0, rows1)
    x_v = (xb0, xb1)
    sem_p = (sp0, sp1)
    sem_x = (sx0, sx1)
    sem_r = (sr0, sr1)
    sem_o = (so0, so1)

    def node0_of(t):
        return jnp.minimum(wid + t * NW, nchunks - 1) * CH

    def issue_in(t, b):
        n0 = node0_of(t)
        cp_p = pltpu.async_copy(hists_hbm.at[:, pl.ds(n0, CH)], part_v[b], sem_p[b])
        cp_x = pltpu.async_copy(x_hbm.at[pl.ds(n0, CH)], x_v[b], sem_x[b])
        return cp_p, cp_x

    cps = {("in", 0): issue_in(0, 0)}
    for t in range(nct):
        b = t & 1
        if t + 1 < nct:
            if t >= 1:
                cps[("out", 1 - b)].wait()
            cps[("in", 1 - b)] = issue_in(t + 1, 1 - b)
        cp_p, cp_x = cps[("in", b)]
        cp_p.wait()
        part = part_v[b]
        deg = deg_v[b]

        @plsc.parallel_loop(0, CH // LANES, unroll=2)
        def _merge(j):
            vals = [part[p, pl.ds(j * LANES, LANES)] for p in range(NW)]
            while len(vals) > 1:
                vals = [a + b for a, b in zip(vals[::2], vals[1::2])]
            deg[pl.ds(j * LANES, LANES)] = jnp.minimum(vals[0], num_nodes - 1)
        g0 = pltpu.async_copy(
            emb_hbm.at[deg.at[pl.ds(0, HALF)]], rows_v[b].at[pl.ds(0, HALF)], sem_r[b])
        g1 = pltpu.async_copy(
            emb_hbm.at[deg.at[pl.ds(HALF, HALF)]], rows_v[b].at[pl.ds(HALF, HALF)], sem_r[b])
        cp_x.wait()
        g0.wait()
        g1.wait()
        xb = x_v[b]
        rows = rows_v[b]

        @plsc.parallel_loop(0, CH, unroll=2)
        def _add_row(r):
            for k in range(128 // LANES):
                xb[r, pl.ds(k * LANES, LANES)] = (
                    xb[r, pl.ds(k * LANES, LANES)]
                    + rows[r, pl.ds(k * LANES, LANES)]
                )
        cps[("out", b)] = pltpu.async_copy(
            xb, out_hbm.at[pl.ds(node0_of(t), CH)], sem_o[b])
    cps[("out", 0)].wait()
    cps[("out", 1)].wait()


@jax.jit
def kernel(x, edge_index, degree_embedding):
    num_nodes, node_dim = x.shape
    n_edges = edge_index.shape[1]
    assert node_dim == 128 and num_nodes % CH == 0
    assert n_edges % (NW * IDX_CHUNK) == 0

    dst = edge_index[1]
    mesh = plsc.VectorSubcoreMesh(core_axis_name="c", subcore_axis_name="s")
    params = pltpu.CompilerParams(
        needs_layout_passes=False, use_tc_tiling_on_sc=False)

    hist_call = pl.kernel(
        functools.partial(_hist_body, n_edges, num_nodes),
        out_type=jax.ShapeDtypeStruct((NW, num_nodes), jnp.int32),
        mesh=mesh,
        scratch_types=[
            pltpu.VMEM((num_nodes,), jnp.int32),
            pltpu.VMEM((IDX_CHUNK,), jnp.int32),
            pltpu.VMEM((IDX_CHUNK,), jnp.int32),
            pltpu.SemaphoreType.DMA,
            pltpu.SemaphoreType.DMA,
        ],
        compiler_params=params,
    )
    hists = hist_call(dst)

    gather_call = pl.kernel(
        functools.partial(_gather_body, num_nodes),
        out_type=jax.ShapeDtypeStruct((num_nodes, node_dim), jnp.float32),
        mesh=mesh,
        scratch_types=(
            [pltpu.VMEM((NW, CH), jnp.int32)] * 2
            + [pltpu.VMEM((CH,), jnp.int32)] * 2
            + [pltpu.VMEM((CH, node_dim), jnp.float32)] * 4
            + [pltpu.SemaphoreType.DMA] * 8
        ),
        compiler_params=params,
    )
    return gather_call(x, hists, degree_embedding)


# speedup vs baseline: 3.0477x; 2.8488x over previous
import functools

import jax
import jax.numpy as jnp
from jax import lax
from jax.experimental import pallas as pl
from jax.experimental.pallas import tpu as pltpu
from jax.experimental.pallas import tpu_sc as plsc

NC = 2
NS = 16
NW = NC * NS
LANES = 16

CH = 160
HALF = CH // 2
IDX_CHUNK = 10000


def _worker_id():
    return lax.axis_index("s") * NC + lax.axis_index("c")


def _hist_body(n_edges, num_nodes, dst_hbm, hists_hbm,
               hist_v, idx0, idx1, sem0, sem1):
    wid = _worker_id()
    epw = n_edges // NW
    base = wid * epw
    nchk = epw // IDX_CHUNK
    idx_v = (idx0, idx1)
    sems = (sem0, sem1)

    zvec = jnp.zeros((LANES,), jnp.int32)

    @plsc.parallel_loop(0, num_nodes // LANES, unroll=8)
    def _zero(i):
        hist_v[pl.ds(i * LANES, LANES)] = zvec

    ones = jnp.ones((LANES,), jnp.int32)
    cps = {}
    cps[0] = pltpu.async_copy(
        dst_hbm.at[pl.ds(base, IDX_CHUNK)], idx0, sem0)
    for c in range(nchk):
        b = c & 1
        if c + 1 < nchk:
            cps[1 - b] = pltpu.async_copy(
                dst_hbm.at[pl.ds(base + (c + 1) * IDX_CHUNK, IDX_CHUNK)],
                idx_v[1 - b], sems[1 - b])
        cps[b].wait()
        buf = idx_v[b]

        @plsc.parallel_loop(0, IDX_CHUNK // LANES, unroll=8)
        def _acc(j):
            idx = buf[pl.ds(j * LANES, LANES)]
            plsc.addupdate_scatter(hist_v, [idx], ones)

    pltpu.sync_copy(hist_v, hists_hbm.at[wid])


CACHE_ROWS = 128


def _gather_body(num_nodes, x_hbm, hists_hbm, emb_hbm, out_hbm,
                 part0, part1, deg0, deg1, rows, cache_v, xb0, xb1,
                 sp0, sp1, sx0, sx1, sem_r, so0, so1):
    wid = _worker_id()
    nchunks = num_nodes // CH
    nct = nchunks // NW + (1 if nchunks % NW else 0)
    part_v = (part0, part1)
    deg_v = (deg0, deg1)
    x_v = (xb0, xb1)
    sem_p = (sp0, sp1)
    sem_x = (sx0, sx1)
    sem_o = (so0, so1)

    pltpu.sync_copy(emb_hbm.at[pl.ds(0, CACHE_ROWS)], cache_v)

    def node0_of(t):
        return jnp.minimum(wid + t * NW, nchunks - 1) * CH

    def issue_in(t, b):
        n0 = node0_of(t)
        cp_p = pltpu.async_copy(hists_hbm.at[:, pl.ds(n0, CH)], part_v[b], sem_p[b])
        cp_x = pltpu.async_copy(x_hbm.at[pl.ds(n0, CH)], x_v[b], sem_x[b])
        return cp_p, cp_x

    issue_in(0, 0)
    issue_in(1, 1)

    def wait_out(t, b):
        pltpu.make_async_copy(x_v[b], out_hbm.at[pl.ds(node0_of(t), CH)],
                              sem_o[b]).wait()

    def pair_body(u, _):
        for b in range(2):
            t = 2 * u + b
            step_body(t, b)
        return _

    def step_body(t, b):
        pltpu.make_async_copy(
            hists_hbm.at[:, pl.ds(node0_of(t), CH)], part_v[b], sem_p[b]).wait()
        part = part_v[b]
        deg = deg_v[b]

        def merge_body(j, mx):
            vals = [part[p, pl.ds(j * LANES, LANES)] for p in range(NW)]
            while len(vals) > 1:
                vals = [a + b for a, b in zip(vals[::2], vals[1::2])]
            deg[pl.ds(j * LANES, LANES)] = jnp.minimum(vals[0], num_nodes - 1)
            return jnp.maximum(mx, vals[0])

        mxvec = plsc.parallel_loop(
            0, CH // LANES, unroll=2,
            carry=jnp.zeros((LANES,), jnp.int32))(merge_body)
        maxdeg = jnp.max(mxvec)
        pltpu.make_async_copy(
            x_hbm.at[pl.ds(node0_of(t), CH)], x_v[b], sem_x[b]).wait()
        xb = x_v[b]

        @pl.when(maxdeg < CACHE_ROWS)
        def _fast():
            @plsc.parallel_loop(0, CH // LANES, unroll=1)
            def _add_cached(g):
                dvec = deg[pl.ds(g * LANES, LANES)]
                for i in range(LANES):
                    d = dvec[i]
                    r = g * LANES + i
                    for k in range(128 // LANES):
                        xb[r, pl.ds(k * LANES, LANES)] = (
                            xb[r, pl.ds(k * LANES, LANES)]
                            + cache_v[d, pl.ds(k * LANES, LANES)]
                        )

        @pl.when(maxdeg >= CACHE_ROWS)
        def _slow():
            g0 = pltpu.async_copy(
                emb_hbm.at[deg.at[pl.ds(0, HALF)]], rows.at[pl.ds(0, HALF)], sem_r)
            g1 = pltpu.async_copy(
                emb_hbm.at[deg.at[pl.ds(HALF, HALF)]], rows.at[pl.ds(HALF, HALF)], sem_r)
            g0.wait()
            g1.wait()

            @plsc.parallel_loop(0, CH, unroll=2)
            def _add_row(r):
                for k in range(128 // LANES):
                    xb[r, pl.ds(k * LANES, LANES)] = (
                        xb[r, pl.ds(k * LANES, LANES)]
                        + rows[r, pl.ds(k * LANES, LANES)]
                    )

        pltpu.async_copy(xb, out_hbm.at[pl.ds(node0_of(t), CH)], sem_o[b])

        @pl.when(t + 2 < nct)
        def _refill():
            wait_out(t, b)
            issue_in(t + 2, b)

    lax.fori_loop(0, nct // 2, pair_body, None)
    wait_out(nct - 2, 0)
    wait_out(nct - 1, 1)


@jax.jit
def kernel(x, edge_index, degree_embedding):
    num_nodes, node_dim = x.shape
    n_edges = edge_index.shape[1]
    assert node_dim == 128 and num_nodes % CH == 0
    assert n_edges % (NW * IDX_CHUNK) == 0

    dst = edge_index[1]
    mesh = plsc.VectorSubcoreMesh(core_axis_name="c", subcore_axis_name="s")
    params = pltpu.CompilerParams(
        needs_layout_passes=False, use_tc_tiling_on_sc=False)

    hist_call = pl.kernel(
        functools.partial(_hist_body, n_edges, num_nodes),
        out_type=jax.ShapeDtypeStruct((NW, num_nodes), jnp.int32),
        mesh=mesh,
        scratch_types=[
            pltpu.VMEM((num_nodes,), jnp.int32),
            pltpu.VMEM((IDX_CHUNK,), jnp.int32),
            pltpu.VMEM((IDX_CHUNK,), jnp.int32),
            pltpu.SemaphoreType.DMA,
            pltpu.SemaphoreType.DMA,
        ],
        compiler_params=params,
    )
    hists = hist_call(dst)

    gather_call = pl.kernel(
        functools.partial(_gather_body, num_nodes),
        out_type=jax.ShapeDtypeStruct((num_nodes, node_dim), jnp.float32),
        mesh=mesh,
        scratch_types=(
            [pltpu.VMEM((NW, CH), jnp.int32)] * 2
            + [pltpu.VMEM((CH,), jnp.int32)] * 2
            + [pltpu.VMEM((CH, node_dim), jnp.float32)]
            + [pltpu.VMEM((CACHE_ROWS, node_dim), jnp.float32)]
            + [pltpu.VMEM((CH, node_dim), jnp.float32)] * 2
            + [pltpu.SemaphoreType.DMA] * 7
        ),
        compiler_params=params,
    )
    return gather_call(x, hists, degree_embedding)


# speedup vs baseline: 3.0663x; 1.0061x over previous
import functools

import jax
import jax.numpy as jnp
from jax import lax
from jax.experimental import pallas as pl
from jax.experimental.pallas import tpu as pltpu
from jax.experimental.pallas import tpu_sc as plsc

NC = 2
NS = 16
NW = NC * NS
LANES = 16

CH = 160
HALF = CH // 2
IDX_CHUNK = 4000
CACHE_ROWS = 112


def _body(n_edges, num_nodes, dst_hbm, x_hbm, emb_hbm, out_hbm, hists_hbm,
          hist_v, idx0, idx1, part0, part1, deg_v, rows, cache_v, xb0, xb1,
          si0, si1, sp0, sp1, sx0, sx1, sem_r, so0, so1):
    cid = lax.axis_index("c")
    sid = lax.axis_index("s")
    nchunks = num_nodes // CH
    n_sc0 = nchunks // 2
    split = n_sc0 * CH
    hlen = num_nodes - split

    lo = cid * split
    hi = jnp.where(cid == 0, split, num_nodes)
    zvec = jnp.zeros((LANES,), jnp.int32)

    @plsc.parallel_loop(0, hlen // LANES, unroll=8)
    def _zero(i):
        hist_v[pl.ds(i * LANES, LANES)] = zvec

    ones = jnp.ones((LANES,), jnp.int32)
    epw = n_edges // NS
    base = sid * epw
    nchk = epw // IDX_CHUNK
    idx_v = (idx0, idx1)
    sems = (si0, si1)
    cps = {0: pltpu.async_copy(dst_hbm.at[pl.ds(base, IDX_CHUNK)], idx0, si0)}
    for c in range(nchk):
        b = c & 1
        if c + 1 < nchk:
            cps[1 - b] = pltpu.async_copy(
                dst_hbm.at[pl.ds(base + (c + 1) * IDX_CHUNK, IDX_CHUNK)],
                idx_v[1 - b], sems[1 - b])
        cps[b].wait()
        buf = idx_v[b]

        @plsc.parallel_loop(0, IDX_CHUNK // LANES, unroll=8)
        def _acc(j):
            idx = buf[pl.ds(j * LANES, LANES)]
            keep = jnp.logical_and(idx >= lo, idx < hi)
            plsc.addupdate_scatter(hist_v, [idx - lo], ones, mask=keep)

    pltpu.sync_copy(hist_v, hists_hbm.at[cid, sid])
    plsc.subcore_barrier()

    pltpu.sync_copy(emb_hbm.at[pl.ds(0, CACHE_ROWS)], cache_v)

    part_v = (part0, part1)
    x_v = (xb0, xb1)
    sem_p = (sp0, sp1)
    sem_x = (sx0, sx1)
    sem_o = (so0, so1)
    nsc_last = jnp.where(cid == 0, n_sc0 - 1, nchunks - n_sc0 - 1)
    cbase = cid * n_sc0
    nct = 20
    assert (max(n_sc0, nchunks - n_sc0) + NS - 1) // NS <= nct

    def node0_of(t):
        l = jnp.minimum(sid + t * NS, nsc_last)
        return (cbase + l) * CH, l * CH

    def issue_in(t, b):
        n0, off = node0_of(t)
        pltpu.async_copy(hists_hbm.at[cid, :, pl.ds(off, CH)], part_v[b], sem_p[b])
        pltpu.async_copy(x_hbm.at[pl.ds(n0, CH)], x_v[b], sem_x[b])

    issue_in(0, 0)
    issue_in(1, 1)

    def wait_out(t, b):
        n0, _ = node0_of(t)
        pltpu.make_async_copy(x_v[b], out_hbm.at[pl.ds(n0, CH)], sem_o[b]).wait()

    def step_body(t, b):
        n0, off = node0_of(t)
        pltpu.make_async_copy(
            hists_hbm.at[cid, :, pl.ds(off, CH)], part_v[b], sem_p[b]).wait()
        part = part_v[b]

        def merge_body(j, mx):
            vals = [part[p, pl.ds(j * LANES, LANES)] for p in range(NS)]
            while len(vals) > 1:
                vals = [v0 + v1 for v0, v1 in zip(vals[::2], vals[1::2])]
            deg_v[pl.ds(j * LANES, LANES)] = jnp.minimum(vals[0], num_nodes - 1)
            return jnp.maximum(mx, vals[0])

        mxvec = plsc.parallel_loop(
            0, CH // LANES, unroll=2,
            carry=jnp.zeros((LANES,), jnp.int32))(merge_body)
        maxdeg = jnp.max(mxvec)
        pltpu.make_async_copy(x_hbm.at[pl.ds(n0, CH)], x_v[b], sem_x[b]).wait()
        xb = x_v[b]

        @pl.when(maxdeg < CACHE_ROWS)
        def _fast():
            @plsc.parallel_loop(0, CH // LANES, unroll=1)
            def _add_cached(g):
                dvec = deg_v[pl.ds(g * LANES, LANES)]
                for i in range(LANES):
                    d = dvec[i]
                    r = g * LANES + i
                    for k in range(128 // LANES):
                        xb[r, pl.ds(k * LANES, LANES)] = (
                            xb[r, pl.ds(k * LANES, LANES)]
                            + cache_v[d, pl.ds(k * LANES, LANES)]
                        )

        @pl.when(maxdeg >= CACHE_ROWS)
        def _slow():
            for h in range(2):
                pltpu.async_copy(
                    emb_hbm.at[deg_v.at[pl.ds(h * HALF, HALF)]], rows, sem_r
                ).wait()

                @plsc.parallel_loop(0, HALF, unroll=2)
                def _add_row(r):
                    for k in range(128 // LANES):
                        xb[h * HALF + r, pl.ds(k * LANES, LANES)] = (
                            xb[h * HALF + r, pl.ds(k * LANES, LANES)]
                            + rows[r, pl.ds(k * LANES, LANES)]
                        )

        pltpu.async_copy(xb, out_hbm.at[pl.ds(n0, CH)], sem_o[b])

        @pl.when(t + 2 < nct)
        def _refill():
            wait_out(t, b)
            issue_in(t + 2, b)

    def pair_body(u, _):
        for b in range(2):
            step_body(2 * u + b, b)
        return _

    lax.fori_loop(0, nct // 2, pair_body, None)
    wait_out(nct - 2, 0)
    wait_out(nct - 1, 1)


@jax.jit
def kernel(x, edge_index, degree_embedding):
    num_nodes, node_dim = x.shape
    n_edges = edge_index.shape[1]
    assert node_dim == 128 and num_nodes % CH == 0
    assert n_edges % (NS * IDX_CHUNK) == 0 and IDX_CHUNK % LANES == 0

    nchunks = num_nodes // CH
    hlen = num_nodes - (nchunks // 2) * CH

    dst = edge_index[1]
    mesh = plsc.VectorSubcoreMesh(core_axis_name="c", subcore_axis_name="s")
    params = pltpu.CompilerParams(
        needs_layout_passes=False, use_tc_tiling_on_sc=False)

    call = pl.kernel(
        functools.partial(_body, n_edges, num_nodes),
        out_type=(
            jax.ShapeDtypeStruct((num_nodes, node_dim), jnp.float32),
            jax.ShapeDtypeStruct((NC, NS, hlen), jnp.int32),
        ),
        mesh=mesh,
        scratch_types=(
            [pltpu.VMEM((hlen,), jnp.int32)]
            + [pltpu.VMEM((IDX_CHUNK,), jnp.int32)] * 2
            + [pltpu.VMEM((NS, CH), jnp.int32)] * 2
            + [pltpu.VMEM((CH,), jnp.int32)]
            + [pltpu.VMEM((HALF, node_dim), jnp.float32)]
            + [pltpu.VMEM((CACHE_ROWS, node_dim), jnp.float32)]
            + [pltpu.VMEM((CH, node_dim), jnp.float32)] * 2
            + [pltpu.SemaphoreType.DMA] * 9
        ),
        compiler_params=params,
    )
    out, _ = call(dst, x, degree_embedding)
    return out


# speedup vs baseline: 3.0794x; 1.0043x over previous
import functools

import jax
import jax.numpy as jnp
from jax import lax
from jax.experimental import pallas as pl
from jax.experimental.pallas import tpu as pltpu
from jax.experimental.pallas import tpu_sc as plsc

NC = 2
NS = 16
NW = NC * NS
LANES = 16

CH = 160
HALF = CH // 2
IDX_CHUNK = 4000
CACHE_ROWS = 112


def _phase1(n_edges, num_nodes, dst_hbm, hists_hbm, cid, sid,
            si0, si1, hist_v, idx0, idx1):
    nchunks = num_nodes // CH
    split = (nchunks // 2) * CH
    hlen = num_nodes - split
    lo = cid * split
    hi = jnp.where(cid == 0, split, num_nodes)
    zvec = jnp.zeros((LANES,), jnp.int32)

    @plsc.parallel_loop(0, hlen // LANES, unroll=8)
    def _zero(i):
        hist_v[pl.ds(i * LANES, LANES)] = zvec

    ones = jnp.ones((LANES,), jnp.int32)
    epw = n_edges // NS
    base = sid * epw
    nchk = epw // IDX_CHUNK
    idx_v = (idx0, idx1)
    sems = (si0, si1)
    cps = {0: pltpu.async_copy(dst_hbm.at[pl.ds(base, IDX_CHUNK)], idx0, si0)}
    for c in range(nchk):
        b = c & 1
        if c + 1 < nchk:
            cps[1 - b] = pltpu.async_copy(
                dst_hbm.at[pl.ds(base + (c + 1) * IDX_CHUNK, IDX_CHUNK)],
                idx_v[1 - b], sems[1 - b])
        cps[b].wait()
        buf = idx_v[b]

        @plsc.parallel_loop(0, IDX_CHUNK // LANES, unroll=8)
        def _acc(j):
            idx = buf[pl.ds(j * LANES, LANES)]
            keep = jnp.logical_and(idx >= lo, idx < hi)
            plsc.addupdate_scatter(hist_v, [idx - lo], ones, mask=keep)

    pltpu.sync_copy(hist_v, hists_hbm.at[cid, sid])


def _phase2(num_nodes, x_hbm, emb_hbm, out_hbm, hists_hbm, cid, sid, deg_v,
            sp0, sp1, sx0, sx1, sem_r, so0, so1,
            part0, part1, rows, cache_v, xb0, xb1, ob0, ob1):
    nchunks = num_nodes // CH
    n_sc0 = nchunks // 2
    pltpu.sync_copy(emb_hbm.at[pl.ds(0, CACHE_ROWS)], cache_v)

    part_v = (part0, part1)
    x_v = (xb0, xb1)
    o_v = (ob0, ob1)
    sem_p = (sp0, sp1)
    sem_x = (sx0, sx1)
    sem_o = (so0, so1)
    nsc_last = jnp.where(cid == 0, n_sc0 - 1, nchunks - n_sc0 - 1)
    cbase = cid * n_sc0
    nct = 20
    assert (max(n_sc0, nchunks - n_sc0) + NS - 1) // NS <= nct

    def node0_of(t):
        l = jnp.minimum(sid + t * NS, nsc_last)
        return (cbase + l) * CH, l * CH

    def issue_in(t, b):
        n0, off = node0_of(t)
        pltpu.async_copy(hists_hbm.at[cid, :, pl.ds(off, CH)], part_v[b], sem_p[b])
        pltpu.async_copy(x_hbm.at[pl.ds(n0, CH)], x_v[b], sem_x[b])

    issue_in(0, 0)
    issue_in(1, 1)

    def wait_out(t, b):
        n0, _ = node0_of(t)
        pltpu.make_async_copy(o_v[b], out_hbm.at[pl.ds(n0, CH)], sem_o[b]).wait()

    def step_body(t, b):
        n0, off = node0_of(t)
        pltpu.make_async_copy(
            hists_hbm.at[cid, :, pl.ds(off, CH)], part_v[b], sem_p[b]).wait()
        part = part_v[b]

        def merge_body(j, mx):
            vals = [part[p, pl.ds(j * LANES, LANES)] for p in range(NS)]
            while len(vals) > 1:
                vals = [v0 + v1 for v0, v1 in zip(vals[::2], vals[1::2])]
            deg_v[pl.ds(j * LANES, LANES)] = jnp.minimum(vals[0], num_nodes - 1)
            return jnp.maximum(mx, vals[0])

        mxvec = plsc.parallel_loop(
            0, CH // LANES, unroll=2,
            carry=jnp.zeros((LANES,), jnp.int32))(merge_body)
        maxdeg = jnp.max(mxvec)
        pltpu.make_async_copy(x_hbm.at[pl.ds(n0, CH)], x_v[b], sem_x[b]).wait()

        @pl.when(t >= 2)
        def _drain():
            wait_out(t - 2, b)

        xb = x_v[b]
        ob = o_v[b]

        @pl.when(maxdeg < CACHE_ROWS)
        def _fast():
            @plsc.parallel_loop(0, CH // LANES, unroll=2)
            def _add_cached(g):
                dvec = deg_v[pl.ds(g * LANES, LANES)]
                for i in range(LANES):
                    d = dvec[i]
                    r = g * LANES + i
                    for k in range(128 // LANES):
                        ob[r, pl.ds(k * LANES, LANES)] = (
                            xb[r, pl.ds(k * LANES, LANES)]
                            + cache_v[d, pl.ds(k * LANES, LANES)]
                        )

        @pl.when(maxdeg >= CACHE_ROWS)
        def _slow():
            for h in range(2):
                pltpu.async_copy(
                    emb_hbm.at[deg_v.at[pl.ds(h * HALF, HALF)]], rows, sem_r
                ).wait()

                @plsc.parallel_loop(0, HALF, unroll=2)
                def _add_row(r):
                    for k in range(128 // LANES):
                        ob[h * HALF + r, pl.ds(k * LANES, LANES)] = (
                            xb[h * HALF + r, pl.ds(k * LANES, LANES)]
                            + rows[r, pl.ds(k * LANES, LANES)]
                        )

        pltpu.async_copy(ob, out_hbm.at[pl.ds(n0, CH)], sem_o[b])

        @pl.when(t + 2 < nct)
        def _refill():
            issue_in(t + 2, b)

    def pair_body(u, _):
        for b in range(2):
            step_body(2 * u + b, b)
        return _

    lax.fori_loop(0, nct // 2, pair_body, None)
    wait_out(nct - 2, 0)
    wait_out(nct - 1, 1)


def _body(n_edges, num_nodes, dst_hbm, x_hbm, emb_hbm, out_hbm, hists_hbm,
          deg_v, si0, si1, sp0, sp1, sx0, sx1, sem_r, so0, so1):
    cid = lax.axis_index("c")
    sid = lax.axis_index("s")
    nchunks = num_nodes // CH
    hlen = num_nodes - (nchunks // 2) * CH

    pl.run_scoped(
        functools.partial(_phase1, n_edges, num_nodes, dst_hbm, hists_hbm,
                          cid, sid, si0, si1),
        pltpu.VMEM((hlen,), jnp.int32),
        pltpu.VMEM((IDX_CHUNK,), jnp.int32),
        pltpu.VMEM((IDX_CHUNK,), jnp.int32),
    )
    plsc.subcore_barrier()
    pl.run_scoped(
        functools.partial(_phase2, num_nodes, x_hbm, emb_hbm, out_hbm,
                          hists_hbm, cid, sid, deg_v,
                          sp0, sp1, sx0, sx1, sem_r, so0, so1),
        pltpu.VMEM((NS, CH), jnp.int32),
        pltpu.VMEM((NS, CH), jnp.int32),
        pltpu.VMEM((HALF, 128), jnp.float32),
        pltpu.VMEM((CACHE_ROWS, 128), jnp.float32),
        pltpu.VMEM((CH, 128), jnp.float32),
        pltpu.VMEM((CH, 128), jnp.float32),
        pltpu.VMEM((CH, 128), jnp.float32),
        pltpu.VMEM((CH, 128), jnp.float32),
    )


@jax.jit
def kernel(x, edge_index, degree_embedding):
    num_nodes, node_dim = x.shape
    n_edges = edge_index.shape[1]
    assert node_dim == 128 and num_nodes % CH == 0
    assert n_edges % (NS * IDX_CHUNK) == 0 and IDX_CHUNK % LANES == 0

    nchunks = num_nodes // CH
    hlen = num_nodes - (nchunks // 2) * CH

    dst = edge_index[1]
    mesh = plsc.VectorSubcoreMesh(core_axis_name="c", subcore_axis_name="s")
    params = pltpu.CompilerParams(
        needs_layout_passes=False, use_tc_tiling_on_sc=False)

    call = pl.kernel(
        functools.partial(_body, n_edges, num_nodes),
        out_type=(
            jax.ShapeDtypeStruct((num_nodes, node_dim), jnp.float32),
            jax.ShapeDtypeStruct((NC, NS, hlen), jnp.int32),
        ),
        mesh=mesh,
        scratch_types=(
            [pltpu.VMEM((CH,), jnp.int32)]
            + [pltpu.SemaphoreType.DMA] * 9
        ),
        compiler_params=params,
    )
    out, _ = call(dst, x, degree_embedding)
    return out
